# baseline (speedup 1.0000x reference)
import jax
import jax.numpy as jnp
from jax import lax
from jax.experimental import pallas as pl
from jax.experimental.pallas import tpu as pltpu
from jax.experimental.pallas import tpu_sc as plsc

SEQ_LEN = 8192
OUT_DIM = 1024
BATCH = 4
BLOCK_SEQ = 512

NC = 2
NS = 16
NW = NC * NS
LANES = 16

TOTAL_WORDS = BATCH * SEQ_LEN * OUT_DIM
POS_WORDS = SEQ_LEN * OUT_DIM
WORDS_PER_WORKER = TOTAL_WORDS // NW
CHUNK_WORDS = 32 * 1024
N_CHUNKS = WORDS_PER_WORKER // CHUNK_WORDS
VECS_PER_CHUNK = CHUNK_WORDS // LANES


def _sc_body(in_hbm, pos_hbm, out_hbm, buf_in, buf_pos, sem_i, sem_p, sem_o):
    wid = lax.axis_index("s") * NC + lax.axis_index("c")
    base = wid * WORDS_PER_WORKER
    pos_base = (wid % (POS_WORDS // WORDS_PER_WORKER)) * WORDS_PER_WORKER

    def chunk_body(c, _):
        off = c * CHUNK_WORDS
        cp_i = pltpu.async_copy(
            in_hbm.at[pl.ds(base + off, CHUNK_WORDS)], buf_in, sem_i)
        cp_p = pltpu.async_copy(
            pos_hbm.at[pl.ds(pos_base + off, CHUNK_WORDS)], buf_pos, sem_p)
        cp_i.wait()
        cp_p.wait()

        @plsc.parallel_loop(0, VECS_PER_CHUNK, 1, unroll=8)
        def vec_body(j):
            s = pl.ds(j * LANES, LANES)
            plsc.addupdate(buf_in.at[s], buf_pos[s])
        pltpu.async_copy(
            buf_in, out_hbm.at[pl.ds(base + off, CHUNK_WORDS)], sem_o).wait()
        return ()

    lax.fori_loop(0, N_CHUNKS, chunk_body, ())


def _sc_kernel(inputs_flat, pos_flat):
    mesh = plsc.VectorSubcoreMesh(core_axis_name="c", subcore_axis_name="s")
    return pl.kernel(
        _sc_body,
        out_type=jax.ShapeDtypeStruct((TOTAL_WORDS,), jnp.float32),
        mesh=mesh,
        scratch_types=[
            pltpu.VMEM((CHUNK_WORDS,), jnp.float32),
            pltpu.VMEM((CHUNK_WORDS,), jnp.float32),
            pltpu.SemaphoreType.DMA,
            pltpu.SemaphoreType.DMA,
            pltpu.SemaphoreType.DMA,
        ],
    )(inputs_flat, pos_flat)


def kernel(inputs, pos_table):
    out = _sc_kernel(inputs.reshape(-1), pos_table.reshape(-1))
    return out.reshape(BATCH, SEQ_LEN, OUT_DIM)


# speedup vs baseline: 1.1894x; 1.1894x over previous
import jax
import jax.numpy as jnp
from jax import lax
from jax.experimental import pallas as pl
from jax.experimental.pallas import tpu as pltpu
from jax.experimental.pallas import tpu_sc as plsc

SEQ_LEN = 8192
OUT_DIM = 1024
BATCH = 4
BLOCK_SEQ = 512

NC = 2
NS = 16
NW = NC * NS
LANES = 16

TOTAL_WORDS = BATCH * SEQ_LEN * OUT_DIM
POS_WORDS = SEQ_LEN * OUT_DIM
WORDS_PER_WORKER = TOTAL_WORDS // NW
CHUNK_WORDS = 16 * 1024
N_CHUNKS = WORDS_PER_WORKER // CHUNK_WORDS
VECS_PER_CHUNK = CHUNK_WORDS // LANES


def _sc_body(in_hbm, pos_hbm, out_hbm,
             in0, in1, pos0, pos1, o0, o1,
             si0, si1, sp0, sp1, so0, so1):
    wid = lax.axis_index("s") * NC + lax.axis_index("c")
    base = wid * WORDS_PER_WORKER
    pos_base = (wid % (POS_WORDS // WORDS_PER_WORKER)) * WORDS_PER_WORKER

    bufs = ((in0, pos0, o0, si0, sp0, so0),
            (in1, pos1, o1, si1, sp1, so1))

    def start_loads(c, bi, bp, s_i, s_p):
        off = c * CHUNK_WORDS
        pltpu.async_copy(in_hbm.at[pl.ds(base + off, CHUNK_WORDS)], bi, s_i)
        pltpu.async_copy(pos_hbm.at[pl.ds(pos_base + off, CHUNK_WORDS)], bp, s_p)

    def process(c, bi, bp, bo, s_i, s_p, s_o):
        off = c * CHUNK_WORDS
        pltpu.make_async_copy(in_hbm.at[pl.ds(base + off, CHUNK_WORDS)], bi, s_i).wait()
        pltpu.make_async_copy(pos_hbm.at[pl.ds(pos_base + off, CHUNK_WORDS)], bp, s_p).wait()

        @pl.when(c >= 2)
        def _():
            pltpu.make_async_copy(bo, out_hbm.at[pl.ds(base + off, CHUNK_WORDS)], s_o).wait()

        @plsc.parallel_loop(0, VECS_PER_CHUNK, 1, unroll=8)
        def _(j):
            s = pl.ds(j * LANES, LANES)
            bo[s] = bi[s] + bp[s]

        pltpu.async_copy(bo, out_hbm.at[pl.ds(base + off, CHUNK_WORDS)], s_o)

        @pl.when(c + 2 < N_CHUNKS)
        def _():
            start_loads(c + 2, bi, bp, s_i, s_p)

    start_loads(0, in0, pos0, si0, sp0)
    start_loads(1, in1, pos1, si1, sp1)

    def chunk_pair(k, _):
        c = k * 2
        bi, bp, bo, s_i, s_p, s_o = bufs[0]
        process(c, bi, bp, bo, s_i, s_p, s_o)
        bi, bp, bo, s_i, s_p, s_o = bufs[1]
        process(c + 1, bi, bp, bo, s_i, s_p, s_o)
        return ()

    lax.fori_loop(0, N_CHUNKS // 2, chunk_pair, ())

    pltpu.make_async_copy(o0, out_hbm.at[pl.ds(base, CHUNK_WORDS)], so0).wait()
    pltpu.make_async_copy(o1, out_hbm.at[pl.ds(base, CHUNK_WORDS)], so1).wait()


def _sc_kernel(inputs_flat, pos_flat):
    mesh = plsc.VectorSubcoreMesh(core_axis_name="c", subcore_axis_name="s")
    buf = lambda: pltpu.VMEM((CHUNK_WORDS,), jnp.float32)
    return pl.kernel(
        _sc_body,
        out_type=jax.ShapeDtypeStruct((TOTAL_WORDS,), jnp.float32),
        mesh=mesh,
        scratch_types=[buf(), buf(), buf(), buf(), buf(), buf()]
        + [pltpu.SemaphoreType.DMA] * 6,
    )(inputs_flat, pos_flat)


def kernel(inputs, pos_table):
    out = _sc_kernel(inputs.reshape(-1), pos_table.reshape(-1))
    return out.reshape(BATCH, SEQ_LEN, OUT_DIM)


# speedup vs baseline: 4.6430x; 3.9036x over previous
import jax
import jax.numpy as jnp
from jax.experimental import pallas as pl

SEQ_LEN = 8192
OUT_DIM = 1024
BATCH = 4
BLOCK_SEQ = 512


def _add_kernel(in_ref, pos_ref, out_ref):
    out_ref[...] = in_ref[...] + pos_ref[...][None, :, :]


def kernel(inputs, pos_table):
    n_seq = SEQ_LEN // BLOCK_SEQ
    return pl.pallas_call(
        _add_kernel,
        grid=(n_seq, BATCH),
        in_specs=[
            pl.BlockSpec((1, BLOCK_SEQ, OUT_DIM), lambda i, b: (b, i, 0)),
            pl.BlockSpec((BLOCK_SEQ, OUT_DIM), lambda i, b: (i, 0)),
        ],
        out_specs=pl.BlockSpec((1, BLOCK_SEQ, OUT_DIM), lambda i, b: (b, i, 0)),
        out_shape=jax.ShapeDtypeStruct((BATCH, SEQ_LEN, OUT_DIM), inputs.dtype),
    )(inputs, pos_table)


# speedup vs baseline: 5.3924x; 1.1614x over previous
import jax
import jax.numpy as jnp
from jax.experimental import pallas as pl

SEQ_LEN = 8192
OUT_DIM = 1024
BATCH = 4
BLOCK_SEQ = 1024


def _add_kernel(in_ref, pos_ref, out_ref):
    out_ref[...] = in_ref[...] + pos_ref[...][None, :, :]


def kernel(inputs, pos_table):
    n_seq = SEQ_LEN // BLOCK_SEQ
    return pl.pallas_call(
        _add_kernel,
        grid=(n_seq, BATCH // 2),
        in_specs=[
            pl.BlockSpec((2, BLOCK_SEQ, OUT_DIM), lambda i, b: (b, i, 0)),
            pl.BlockSpec((BLOCK_SEQ, OUT_DIM), lambda i, b: (i, 0)),
        ],
        out_specs=pl.BlockSpec((2, BLOCK_SEQ, OUT_DIM), lambda i, b: (b, i, 0)),
        out_shape=jax.ShapeDtypeStruct((BATCH, SEQ_LEN, OUT_DIM), inputs.dtype),
    )(inputs, pos_table)
